# baseline (speedup 1.0000x reference)
import functools

import jax
import jax.numpy as jnp
import numpy as np
from jax import lax
from jax.experimental import pallas as pl
from jax.experimental.pallas import tpu as pltpu
from jax.experimental.pallas import tpu_sc as plsc

L = 16
NC, NS = 2, 16
NW = NC * NS
B, N = 4, 262144
NPTS = B * N
PPW = NPTS // NW
K = 1024
NCH = PPW // K
NIDX = 512
NG = K // NIDX
VOLSZ = 128 * 128 * 128
NTOT = B * VOLSZ
NT16 = NTOT // 16

CLIP_LO = np.float32(0.001)
CLIP_HI = np.float32(128.0) - np.float32(1.001)

DOFF = (0, 8, 1024, 1032)


BC = 16384
BCR = BC // 16
EPW = NTOT // NW
BNCH = EPW // BC

_mesh = plsc.VectorSubcoreMesh(core_axis_name="c", subcore_axis_name="s")
_params = pltpu.CompilerParams(
    needs_layout_passes=False, use_tc_tiling_on_sc=False
)


@functools.partial(
    pl.kernel,
    mesh=_mesh,
    out_type=jax.ShapeDtypeStruct((2 * NT16, 16), jnp.float32),
    scratch_types=[
        pltpu.VMEM((BC + 16,), jnp.float32),
        pltpu.VMEM((BCR, 16), jnp.float32),
        pltpu.VMEM((BCR, 16), jnp.float32),
    ],
    compiler_params=_params,
)
def _build(vpad, table, buf1, bufe, bufs):
    cid = lax.axis_index("c")
    sid = lax.axis_index("s")
    wid = sid * NC + cid
    e0 = wid * EPW

    def chunk(ch, carry):
        a = e0 + ch * BC
        pltpu.sync_copy(vpad.at[pl.ds(a, BC + 16)], buf1)

        def rows(j, c2):
            bufe[j, :] = buf1[pl.ds(j * 16, 16)]
            bufs[j, :] = buf1[pl.ds(j * 16 + 8, 16)]
            return c2

        lax.fori_loop(0, BCR, rows, 0)
        r0 = a // 16
        pltpu.sync_copy(bufe, table.at[pl.ds(r0, BCR)])
        pltpu.sync_copy(bufs, table.at[pl.ds(NT16 + r0, BCR)])
        return carry

    lax.fori_loop(0, BNCH, chunk, 0)


_scratch = (
    [pltpu.VMEM((K,), jnp.float32) for _ in range(3)]
    + [pltpu.VMEM((K,), jnp.int32) for _ in range(4)]
    + [pltpu.VMEM((K,), jnp.int32)]
    + [pltpu.VMEM((K, 16), jnp.float32) for _ in range(4)]
    + [pltpu.VMEM((K,), jnp.float32) for _ in range(6)]
    + [pltpu.VMEM((K,), jnp.float32)]
    + [pltpu.SemaphoreType.DMA]
)


@functools.partial(
    pl.kernel,
    mesh=_mesh,
    out_type=jax.ShapeDtypeStruct((NPTS,), jnp.float32),
    scratch_types=_scratch,
    compiler_params=_params,
)
def _warp(table, gx, gy, gz, out, *refs):
    grid = refs[0:3]
    idx_s = refs[3:7]
    off_s = refs[7]
    g_s = refs[8:12]
    w_s = refs[12:18]
    o_s = refs[18]
    sem_g = refs[19]
    gin = (gx, gy, gz)

    cid = lax.axis_index("c")
    sid = lax.axis_index("s")
    wid = sid * NC + cid
    base0 = wid * PPW
    vbase = (wid // (NW // B)) * VOLSZ

    lanes = lax.iota(jnp.int32, L)

    def chunk_body(ch, carry):
        base = base0 + ch * K
        for a in range(3):
            pltpu.sync_copy(gin[a].at[pl.ds(base, K)], grid[a])

        def gen(i, c2):
            sl = pl.ds(i * L, L)

            def axis(a):
                t = grid[a][sl] * 128.0
                t = jnp.minimum(jnp.maximum(t, CLIP_LO), CLIP_HI)
                i1 = t.astype(jnp.int32)
                f1 = i1.astype(jnp.float32)
                w = t - f1
                up = jnp.where(w > 0.0, 1.0, 0.0).astype(jnp.float32)
                w2 = (f1 + up) - t
                return i1, w, w2

            ix, wx, wx2 = axis(0)
            iy, wy, wy2 = axis(1)
            iz, wz, wz2 = axis(2)
            colrow = (vbase + ix * 16384 + iy * 128) >> 4
            sel = (iz & 15) == 15
            row = jnp.where(
                sel, (NT16 + ((iz - 8) >> 4)) + colrow, (iz >> 4) + colrow
            )
            off_s[sl] = jnp.where(sel, 7, iz & 15)
            for c in range(4):
                idx_s[c][sl] = row + DOFF[c]
            for a, w in enumerate((wx, wx2, wy, wy2, wz, wz2)):
                w_s[a][sl] = w
            return c2

        lax.fori_loop(0, K // L, gen, 0)

        copies = []
        for c in range(4):
            for j in range(NG):
                copies.append(
                    pltpu.async_copy(
                        table.at[idx_s[c].at[pl.ds(j * NIDX, NIDX)]],
                        g_s[c].at[pl.ds(j * NIDX, NIDX)],
                        sem_g,
                    )
                )
        for cp in copies:
            cp.wait()

        def blend(i, c2):
            sl = pl.ds(i * L, L)
            row = i * L + lanes
            off = off_s[sl]
            off2 = off + 1
            wx = w_s[0][sl]
            wx2 = w_s[1][sl]
            wy = w_s[2][sl]
            wy2 = w_s[3][sl]
            wz = w_s[4][sl]
            wz2 = w_s[5][sl]

            def q(c, o):
                return plsc.load_gather(g_s[c], [row, o])

            lx1 = q(2, off) * wx + q(0, off) * wx2
            lx2 = q(3, off) * wx + q(1, off) * wx2
            ly1 = lx2 * wy + lx1 * wy2
            lx1b = q(2, off2) * wx + q(0, off2) * wx2
            lx2b = q(3, off2) * wx + q(1, off2) * wx2
            ly2 = lx2b * wy + lx1b * wy2
            o_s[sl] = ly2 * wz + ly1 * wz2
            return c2

        lax.fori_loop(0, K // L, blend, 0)
        pltpu.sync_copy(o_s, out.at[pl.ds(base, K)])
        return carry

    lax.fori_loop(0, NCH, chunk_body, 0)


def kernel(image_inputs, image_grid):
    v = image_inputs.reshape(NTOT)
    vpad = jnp.concatenate([v, jnp.zeros((16,), jnp.float32)])
    table = _build(vpad)
    grid_t = jnp.transpose(image_grid, (2, 0, 1)).reshape(3, NPTS)
    out = _warp(table, grid_t[0], grid_t[1], grid_t[2])
    return out.reshape(B, N, 1)


# speedup vs baseline: 1.2176x; 1.2176x over previous
import functools

import jax
import jax.numpy as jnp
import numpy as np
from jax import lax
from jax.experimental import pallas as pl
from jax.experimental.pallas import tpu as pltpu
from jax.experimental.pallas import tpu_sc as plsc

L = 16
NC, NS = 2, 16
NW = NC * NS
B, N = 4, 262144
NPTS = B * N
PPW = NPTS // NW
K = 2048
NCH = PPW // K
NIDX = 512
NG = K // NIDX
VOLSZ = 128 * 128 * 128
NTOT = B * VOLSZ

CLIP_LO = np.float32(0.001)
CLIP_HI = np.float32(128.0) - np.float32(1.001)

OFFS = (0, 16384, 128, 16512, 1, 16385, 129, 16513)

_mesh = plsc.VectorSubcoreMesh(core_axis_name="c", subcore_axis_name="s")

_one_buf = (
    [pltpu.VMEM((K,), jnp.float32) for _ in range(3)]
    + [pltpu.VMEM((K,), jnp.int32) for _ in range(8)]
    + [pltpu.VMEM((K,), jnp.float32) for _ in range(8)]
    + [pltpu.VMEM((K,), jnp.float32) for _ in range(6)]
    + [pltpu.VMEM((K,), jnp.float32)]
    + [pltpu.SemaphoreType.DMA]
)
_scratch = _one_buf + _one_buf
_NB = len(_one_buf)


@functools.partial(
    pl.kernel,
    mesh=_mesh,
    out_type=jax.ShapeDtypeStruct((NPTS,), jnp.float32),
    scratch_types=_scratch,
    compiler_params=pltpu.CompilerParams(
        needs_layout_passes=False, use_tc_tiling_on_sc=False
    ),
)
def _warp(vol, gx, gy, gz, out, *refs):
    bufs = []
    for p in (0, 1):
        r = refs[p * _NB : (p + 1) * _NB]
        bufs.append(
            dict(coords=r[0:3], idx=r[3:11], g=r[11:19], w=r[19:25],
                 o=r[25], sem=r[26])
        )
    gin = (gx, gy, gz)

    cid = lax.axis_index("c")
    sid = lax.axis_index("s")
    wid = sid * NC + cid
    base0 = wid * PPW
    vbase = (wid // (NW // B)) * VOLSZ

    def gen_fire(bf, ch):
        base = base0 + ch * K
        for a in range(3):
            pltpu.sync_copy(gin[a].at[pl.ds(base, K)], bf["coords"][a])

        def gen(i, c2):
            sl = pl.ds(i * L, L)

            def axis(a):
                t = bf["coords"][a][sl] * 128.0
                t = jnp.minimum(jnp.maximum(t, CLIP_LO), CLIP_HI)
                i1 = t.astype(jnp.int32)
                f1 = i1.astype(jnp.float32)
                w = t - f1
                up = jnp.where(w > 0.0, 1.0, 0.0).astype(jnp.float32)
                w2 = (f1 + up) - t
                return i1, w, w2

            ix, wx, wx2 = axis(0)
            iy, wy, wy2 = axis(1)
            iz, wz, wz2 = axis(2)
            ibase = ix * 16384 + iy * 128 + iz + vbase
            for c in range(8):
                bf["idx"][c][sl] = ibase + OFFS[c]
            for a, w in enumerate((wx, wx2, wy, wy2, wz, wz2)):
                bf["w"][a][sl] = w
            return c2

        lax.fori_loop(0, K // L, gen, 0)

        for c in range(8):
            for j in range(NG):
                pltpu.async_copy(
                    vol.at[bf["idx"][c].at[pl.ds(j * NIDX, NIDX)]],
                    bf["g"][c].at[pl.ds(j * NIDX, NIDX)],
                    bf["sem"],
                )

    def finish(bf, ch):
        base = base0 + ch * K
        for c in range(8):
            pltpu.make_async_copy(
                vol.at[pl.ds(0, K)], bf["g"][c], bf["sem"]
            ).wait()

        def blend(i, c2):
            sl = pl.ds(i * L, L)
            g = bf["g"]
            w = bf["w"]
            wx = w[0][sl]
            wx2 = w[1][sl]
            wy = w[2][sl]
            wy2 = w[3][sl]
            wz = w[4][sl]
            wz2 = w[5][sl]
            lx1 = g[1][sl] * wx + g[0][sl] * wx2
            lx2 = g[3][sl] * wx + g[2][sl] * wx2
            ly1 = lx2 * wy + lx1 * wy2
            lx1b = g[5][sl] * wx + g[4][sl] * wx2
            lx2b = g[7][sl] * wx + g[6][sl] * wx2
            ly2 = lx2b * wy + lx1b * wy2
            bf["o"][sl] = ly2 * wz + ly1 * wz2
            return c2

        lax.fori_loop(0, K // L, blend, 0)
        pltpu.sync_copy(bf["o"], out.at[pl.ds(base, K)])

    gen_fire(bufs[0], 0)

    def body(i, carry):
        for par in (0, 1):
            @pl.when((i & 1) == par)
            def _():
                nxt = bufs[1 - par]
                cur = bufs[par]

                @pl.when(i + 1 < NCH)
                def _():
                    gen_fire(nxt, i + 1)

                finish(cur, i)
        return carry

    lax.fori_loop(0, NCH, body, 0)


def kernel(image_inputs, image_grid):
    vol = image_inputs.reshape(NTOT)
    grid_t = jnp.transpose(image_grid, (2, 0, 1)).reshape(3, NPTS)
    out = _warp(vol, grid_t[0], grid_t[1], grid_t[2])
    return out.reshape(B, N, 1)
